# baseline (device time: 106481 ns/iter reference)
import jax
import jax.numpy as jnp
from jax import lax
from jax.experimental import pallas as pl
from jax.experimental.pallas import tpu as pltpu

N_DEV = 32
N_R = 16
N_L = 15
SUBS = 4

P_RING = [0, 3, 4, 7, 15, 12, 11, 8, 16, 19, 20, 23, 31, 28, 27, 24,
          25, 26, 29, 30, 22, 21, 18, 17, 9, 10, 13, 14, 6, 5, 2, 1]
Q_RING = [0, 31, 30, 1, 2, 29, 28, 3, 7, 24, 25, 6, 5, 26, 27, 4,
          8, 23, 22, 9, 10, 21, 20, 11, 15, 16, 17, 14, 13, 18, 19, 12]


def kernel(x, w_mat, scale_x, scale_w):
    m_per, k = x.shape
    _, n_per = w_mat.shape
    m_sub = m_per // SUBS

    def body(x_ref, w_ref, sx_ref, sw_ref, p_ref, q_ref, out_ref,
             xq, wq, rbuf, lbuf, send_r, send_l, recv_r, recv_l):
        my = lax.axis_index("i")
        q = q_ref[my]
        right = p_ref[lax.rem(q + 1, N_DEV)]
        left = p_ref[lax.rem(q + (N_DEV - 1), N_DEV)]

        xq[...] = x_ref[...].astype(jnp.float8_e5m2)

        barrier = pltpu.get_barrier_semaphore()
        for nbr in (left, right):
            pl.semaphore_signal(barrier, inc=1, device_id=(nbr,),
                                device_id_type=pl.DeviceIdType.MESH)
        pl.semaphore_wait(barrier, 2)

        def sub(ref_2d, j):
            return ref_2d.at[pl.ds(j * m_sub, m_sub), :]

        def mk(src, buf, h, j, ssem, rsem, dev):
            return pltpu.make_async_remote_copy(
                src_ref=src, dst_ref=sub(buf.at[h], j),
                send_sem=ssem.at[h, j], recv_sem=rsem.at[h, j],
                device_id=(dev,), device_id_type=pl.DeviceIdType.MESH)

        started = []

        for j in range(SUBS):
            d = mk(sub(xq, j), rbuf, 0, j, send_r, recv_r, right)
            d.start()
            started.append(d)
            d = mk(sub(xq, j), lbuf, 0, j, send_l, recv_l, left)
            d.start()
            started.append(d)

        wq[...] = w_ref[...].astype(jnp.float8_e5m2)
        scale = sx_ref[0, 0] * sw_ref[0, 0]

        def compute(chunk_ref, origin):
            acc = jnp.dot(chunk_ref[...], wq[...],
                          preferred_element_type=jnp.float32)
            y = jnp.maximum(acc * scale, 0.0)
            out_ref[pl.ds(origin * m_per, m_per), :] = y

        compute(xq, my)

        for h in range(N_R):
            for j in range(SUBS):
                mk(sub(rbuf.at[h], j), rbuf, h, j,
                   send_r, recv_r, right).wait_recv()
                if h + 1 < N_R:
                    d = mk(sub(rbuf.at[h], j), rbuf, h + 1, j,
                           send_r, recv_r, right)
                    d.start()
                    started.append(d)
            if h < N_L:
                for j in range(SUBS):
                    mk(sub(lbuf.at[h], j), lbuf, h, j,
                       send_l, recv_l, left).wait_recv()
                    if h + 1 < N_L:
                        d = mk(sub(lbuf.at[h], j), lbuf, h + 1, j,
                               send_l, recv_l, left)
                        d.start()
                        started.append(d)
            compute(rbuf.at[h], p_ref[lax.rem(q + (N_DEV - 1 - h), N_DEV)])
            if h < N_L:
                compute(lbuf.at[h], p_ref[lax.rem(q + (1 + h), N_DEV)])

        for d in started:
            d.wait_send()

    out_shape = jax.ShapeDtypeStruct((N_DEV * m_per, n_per), jnp.float32)
    return pl.pallas_call(
        body,
        out_shape=out_shape,
        in_specs=[
            pl.BlockSpec(memory_space=pltpu.VMEM),
            pl.BlockSpec(memory_space=pltpu.VMEM),
            pl.BlockSpec(memory_space=pltpu.SMEM),
            pl.BlockSpec(memory_space=pltpu.SMEM),
            pl.BlockSpec(memory_space=pltpu.SMEM),
            pl.BlockSpec(memory_space=pltpu.SMEM),
        ],
        out_specs=pl.BlockSpec(memory_space=pltpu.VMEM),
        scratch_shapes=[
            pltpu.VMEM((m_per, k), jnp.float8_e5m2),
            pltpu.VMEM((k, n_per), jnp.float8_e5m2),
            pltpu.VMEM((N_R, m_per, k), jnp.float8_e5m2),
            pltpu.VMEM((N_L, m_per, k), jnp.float8_e5m2),
            pltpu.SemaphoreType.DMA((N_R, SUBS)),
            pltpu.SemaphoreType.DMA((N_L, SUBS)),
            pltpu.SemaphoreType.DMA((N_R, SUBS)),
            pltpu.SemaphoreType.DMA((N_L, SUBS)),
        ],
        compiler_params=pltpu.CompilerParams(
            collective_id=0, vmem_limit_bytes=64 * 1024 * 1024),
    )(x, w_mat, scale_x.reshape(1, 1), scale_w.reshape(1, 1),
      jnp.asarray(P_RING, jnp.int32), jnp.asarray(Q_RING, jnp.int32))


# device time: 105280 ns/iter; 1.0114x vs baseline; 1.0114x over previous
import jax
import jax.numpy as jnp
from jax import lax
from jax.experimental import pallas as pl
from jax.experimental.pallas import tpu as pltpu

N_DEV = 32
N_R = 16
N_L = 15
SUBS = 2

P_RING = [0, 3, 4, 7, 15, 12, 11, 8, 16, 19, 20, 23, 31, 28, 27, 24,
          25, 26, 29, 30, 22, 21, 18, 17, 9, 10, 13, 14, 6, 5, 2, 1]
Q_RING = [0, 31, 30, 1, 2, 29, 28, 3, 7, 24, 25, 6, 5, 26, 27, 4,
          8, 23, 22, 9, 10, 21, 20, 11, 15, 16, 17, 14, 13, 18, 19, 12]


def kernel(x, w_mat, scale_x, scale_w):
    m_per, k = x.shape
    _, n_per = w_mat.shape
    m_sub = m_per // SUBS

    def body(x_ref, w_ref, sx_ref, sw_ref, p_ref, q_ref, out_ref,
             xq, wq, rbuf, lbuf, send_r, send_l, recv_r, recv_l):
        my = lax.axis_index("i")
        q = q_ref[my]
        right = p_ref[lax.rem(q + 1, N_DEV)]
        left = p_ref[lax.rem(q + (N_DEV - 1), N_DEV)]

        xq[...] = x_ref[...].astype(jnp.float8_e5m2)

        barrier = pltpu.get_barrier_semaphore()
        for nbr in (left, right):
            pl.semaphore_signal(barrier, inc=1, device_id=(nbr,),
                                device_id_type=pl.DeviceIdType.MESH)
        pl.semaphore_wait(barrier, 2)

        def sub(ref_2d, j):
            return ref_2d.at[pl.ds(j * m_sub, m_sub), :]

        def mk(src, buf, h, j, ssem, rsem, dev):
            return pltpu.make_async_remote_copy(
                src_ref=src, dst_ref=sub(buf.at[h], j),
                send_sem=ssem.at[h, j], recv_sem=rsem.at[h, j],
                device_id=(dev,), device_id_type=pl.DeviceIdType.MESH)

        started = []

        for j in range(SUBS):
            d = mk(sub(xq, j), rbuf, 0, j, send_r, recv_r, right)
            d.start()
            started.append(d)
            d = mk(sub(xq, j), lbuf, 0, j, send_l, recv_l, left)
            d.start()
            started.append(d)

        wq[...] = w_ref[...].astype(jnp.float8_e5m2)
        scale = sx_ref[0, 0] * sw_ref[0, 0]

        def compute(chunk_ref, origin):
            acc = jnp.dot(chunk_ref[...], wq[...],
                          preferred_element_type=jnp.float32)
            y = jnp.maximum(acc * scale, 0.0)
            out_ref[pl.ds(origin * m_per, m_per), :] = y

        compute(xq, my)

        for h in range(N_R):
            for j in range(SUBS):
                mk(sub(rbuf.at[h], j), rbuf, h, j,
                   send_r, recv_r, right).wait_recv()
                if h + 1 < N_R:
                    d = mk(sub(rbuf.at[h], j), rbuf, h + 1, j,
                           send_r, recv_r, right)
                    d.start()
                    started.append(d)
            if h < N_L:
                for j in range(SUBS):
                    mk(sub(lbuf.at[h], j), lbuf, h, j,
                       send_l, recv_l, left).wait_recv()
                    if h + 1 < N_L:
                        d = mk(sub(lbuf.at[h], j), lbuf, h + 1, j,
                               send_l, recv_l, left)
                        d.start()
                        started.append(d)
            compute(rbuf.at[h], p_ref[lax.rem(q + (N_DEV - 1 - h), N_DEV)])
            if h < N_L:
                compute(lbuf.at[h], p_ref[lax.rem(q + (1 + h), N_DEV)])

        for d in started:
            d.wait_send()

    out_shape = jax.ShapeDtypeStruct((N_DEV * m_per, n_per), jnp.float32)
    return pl.pallas_call(
        body,
        out_shape=out_shape,
        in_specs=[
            pl.BlockSpec(memory_space=pltpu.VMEM),
            pl.BlockSpec(memory_space=pltpu.VMEM),
            pl.BlockSpec(memory_space=pltpu.SMEM),
            pl.BlockSpec(memory_space=pltpu.SMEM),
            pl.BlockSpec(memory_space=pltpu.SMEM),
            pl.BlockSpec(memory_space=pltpu.SMEM),
        ],
        out_specs=pl.BlockSpec(memory_space=pltpu.VMEM),
        scratch_shapes=[
            pltpu.VMEM((m_per, k), jnp.float8_e5m2),
            pltpu.VMEM((k, n_per), jnp.float8_e5m2),
            pltpu.VMEM((N_R, m_per, k), jnp.float8_e5m2),
            pltpu.VMEM((N_L, m_per, k), jnp.float8_e5m2),
            pltpu.SemaphoreType.DMA((N_R, SUBS)),
            pltpu.SemaphoreType.DMA((N_L, SUBS)),
            pltpu.SemaphoreType.DMA((N_R, SUBS)),
            pltpu.SemaphoreType.DMA((N_L, SUBS)),
        ],
        compiler_params=pltpu.CompilerParams(
            collective_id=0, vmem_limit_bytes=64 * 1024 * 1024),
    )(x, w_mat, scale_x.reshape(1, 1), scale_w.reshape(1, 1),
      jnp.asarray(P_RING, jnp.int32), jnp.asarray(Q_RING, jnp.int32))


# device time: 103717 ns/iter; 1.0266x vs baseline; 1.0151x over previous
import jax
import jax.numpy as jnp
from jax import lax
from jax.experimental import pallas as pl
from jax.experimental.pallas import tpu as pltpu

N_DEV = 32
N_R = 16
N_L = 15
SUBS = 2

P_RING = [0, 3, 4, 7, 15, 12, 11, 8, 16, 19, 20, 23, 31, 28, 27, 24,
          25, 26, 29, 30, 22, 21, 18, 17, 9, 10, 13, 14, 6, 5, 2, 1]
Q_RING = [0, 31, 30, 1, 2, 29, 28, 3, 7, 24, 25, 6, 5, 26, 27, 4,
          8, 23, 22, 9, 10, 21, 20, 11, 15, 16, 17, 14, 13, 18, 19, 12]


def kernel(x, w_mat, scale_x, scale_w):
    m_per, k = x.shape
    _, n_per = w_mat.shape
    m_sub = m_per // SUBS

    def body(x_ref, w_ref, sx_ref, sw_ref, p_ref, q_ref, out_ref,
             xf, wf, xq, wq, rbuf, lbuf, load_sems,
             send_r, send_l, recv_r, recv_l):
        my = lax.axis_index("i")
        q = q_ref[my]
        right = p_ref[lax.rem(q + 1, N_DEV)]
        left = p_ref[lax.rem(q + (N_DEV - 1), N_DEV)]

        x_load = pltpu.make_async_copy(x_ref, xf, load_sems.at[0])
        w_load = pltpu.make_async_copy(w_ref, wf, load_sems.at[1])
        x_load.start()
        w_load.start()

        barrier = pltpu.get_barrier_semaphore()
        for nbr in (left, right):
            pl.semaphore_signal(barrier, inc=1, device_id=(nbr,),
                                device_id_type=pl.DeviceIdType.MESH)
        pl.semaphore_wait(barrier, 2)

        x_load.wait()
        xq[...] = xf[...].astype(jnp.float8_e5m2)

        def sub(ref_2d, j):
            return ref_2d.at[pl.ds(j * m_sub, m_sub), :]

        def mk(src, buf, h, j, ssem, rsem, dev):
            return pltpu.make_async_remote_copy(
                src_ref=src, dst_ref=sub(buf.at[h], j),
                send_sem=ssem.at[h, j], recv_sem=rsem.at[h, j],
                device_id=(dev,), device_id_type=pl.DeviceIdType.MESH)

        started = []

        for j in range(SUBS):
            d = mk(sub(xq, j), rbuf, 0, j, send_r, recv_r, right)
            d.start()
            started.append(d)
            d = mk(sub(xq, j), lbuf, 0, j, send_l, recv_l, left)
            d.start()
            started.append(d)

        w_load.wait()
        wq[...] = wf[...].astype(jnp.float8_e5m2)
        scale = sx_ref[0, 0] * sw_ref[0, 0]

        def compute(chunk_ref, origin):
            acc = jnp.dot(chunk_ref[...], wq[...],
                          preferred_element_type=jnp.float32)
            y = jnp.maximum(acc * scale, 0.0)
            out_ref[pl.ds(origin * m_per, m_per), :] = y

        compute(xq, my)

        for h in range(N_R):
            for j in range(SUBS):
                mk(sub(rbuf.at[h], j), rbuf, h, j,
                   send_r, recv_r, right).wait_recv()
                if h + 1 < N_R:
                    d = mk(sub(rbuf.at[h], j), rbuf, h + 1, j,
                           send_r, recv_r, right)
                    d.start()
                    started.append(d)
            if h < N_L:
                for j in range(SUBS):
                    mk(sub(lbuf.at[h], j), lbuf, h, j,
                       send_l, recv_l, left).wait_recv()
                    if h + 1 < N_L:
                        d = mk(sub(lbuf.at[h], j), lbuf, h + 1, j,
                               send_l, recv_l, left)
                        d.start()
                        started.append(d)
            compute(rbuf.at[h], p_ref[lax.rem(q + (N_DEV - 1 - h), N_DEV)])
            if h < N_L:
                compute(lbuf.at[h], p_ref[lax.rem(q + (1 + h), N_DEV)])

        for d in started:
            d.wait_send()

    out_shape = jax.ShapeDtypeStruct((N_DEV * m_per, n_per), jnp.float32)
    return pl.pallas_call(
        body,
        out_shape=out_shape,
        in_specs=[
            pl.BlockSpec(memory_space=pl.ANY),
            pl.BlockSpec(memory_space=pl.ANY),
            pl.BlockSpec(memory_space=pltpu.SMEM),
            pl.BlockSpec(memory_space=pltpu.SMEM),
            pl.BlockSpec(memory_space=pltpu.SMEM),
            pl.BlockSpec(memory_space=pltpu.SMEM),
        ],
        out_specs=pl.BlockSpec(memory_space=pltpu.VMEM),
        scratch_shapes=[
            pltpu.VMEM((m_per, k), jnp.float32),
            pltpu.VMEM((k, n_per), jnp.float32),
            pltpu.VMEM((m_per, k), jnp.float8_e5m2),
            pltpu.VMEM((k, n_per), jnp.float8_e5m2),
            pltpu.VMEM((N_R, m_per, k), jnp.float8_e5m2),
            pltpu.VMEM((N_L, m_per, k), jnp.float8_e5m2),
            pltpu.SemaphoreType.DMA((2,)),
            pltpu.SemaphoreType.DMA((N_R, SUBS)),
            pltpu.SemaphoreType.DMA((N_L, SUBS)),
            pltpu.SemaphoreType.DMA((N_R, SUBS)),
            pltpu.SemaphoreType.DMA((N_L, SUBS)),
        ],
        compiler_params=pltpu.CompilerParams(
            collective_id=0, vmem_limit_bytes=64 * 1024 * 1024),
    )(x, w_mat, scale_x.reshape(1, 1), scale_w.reshape(1, 1),
      jnp.asarray(P_RING, jnp.int32), jnp.asarray(Q_RING, jnp.int32))


# device time: 102513 ns/iter; 1.0387x vs baseline; 1.0117x over previous
import jax
import jax.numpy as jnp
from jax import lax
from jax.experimental import pallas as pl
from jax.experimental.pallas import tpu as pltpu

N_DEV = 32
N_R = 16
N_L = 15
SUBS = 2
N_SLAB = 4

P_RING = [0, 3, 4, 7, 15, 12, 11, 8, 16, 19, 20, 23, 31, 28, 27, 24,
          25, 26, 29, 30, 22, 21, 18, 17, 9, 10, 13, 14, 6, 5, 2, 1]
Q_RING = [0, 31, 30, 1, 2, 29, 28, 3, 7, 24, 25, 6, 5, 26, 27, 4,
          8, 23, 22, 9, 10, 21, 20, 11, 15, 16, 17, 14, 13, 18, 19, 12]


def kernel(x, w_mat, scale_x, scale_w):
    m_per, k = x.shape
    _, n_per = w_mat.shape
    m_sub = m_per // SUBS

    def body(x_ref, w_ref, sx_ref, sw_ref, p_ref, q_ref, out_ref,
             xf, wf, xq, wq, rbuf, lbuf, slab, load_sems, store_sems,
             send_r, send_l, recv_r, recv_l):
        my = lax.axis_index("i")
        q = q_ref[my]
        right = p_ref[lax.rem(q + 1, N_DEV)]
        left = p_ref[lax.rem(q + (N_DEV - 1), N_DEV)]

        x_load = pltpu.make_async_copy(x_ref, xf, load_sems.at[0])
        w_load = pltpu.make_async_copy(w_ref, wf, load_sems.at[1])
        x_load.start()
        w_load.start()

        barrier = pltpu.get_barrier_semaphore()
        for nbr in (left, right):
            pl.semaphore_signal(barrier, inc=1, device_id=(nbr,),
                                device_id_type=pl.DeviceIdType.MESH)
        pl.semaphore_wait(barrier, 2)

        x_load.wait()
        xq[...] = xf[...].astype(jnp.float8_e5m2)

        def sub(ref_2d, j):
            return ref_2d.at[pl.ds(j * m_sub, m_sub), :]

        def mk(src, buf, h, j, ssem, rsem, dev):
            return pltpu.make_async_remote_copy(
                src_ref=src, dst_ref=sub(buf.at[h], j),
                send_sem=ssem.at[h, j], recv_sem=rsem.at[h, j],
                device_id=(dev,), device_id_type=pl.DeviceIdType.MESH)

        started = []

        for j in range(SUBS):
            d = mk(sub(xq, j), rbuf, 0, j, send_r, recv_r, right)
            d.start()
            started.append(d)
            d = mk(sub(xq, j), lbuf, 0, j, send_l, recv_l, left)
            d.start()
            started.append(d)

        w_load.wait()
        wq[...] = wf[...].astype(jnp.float8_e5m2)
        scale = sx_ref[0, 0] * sw_ref[0, 0]

        last_store = [None] * N_SLAB
        n_done = [0]

        def compute(chunk_ref, origin):
            s = n_done[0] % N_SLAB
            n_done[0] += 1
            if last_store[s] is not None:
                last_store[s].wait()
            acc = jnp.dot(chunk_ref[...], wq[...],
                          preferred_element_type=jnp.float32)
            slab[s, :, :] = jnp.maximum(acc * scale, 0.0)
            d = pltpu.make_async_copy(
                slab.at[s], out_ref.at[pl.ds(origin * m_per, m_per), :],
                store_sems.at[s])
            d.start()
            last_store[s] = d

        compute(xq, my)

        for h in range(N_R):
            for j in range(SUBS):
                mk(sub(rbuf.at[h], j), rbuf, h, j,
                   send_r, recv_r, right).wait_recv()
                if h + 1 < N_R:
                    d = mk(sub(rbuf.at[h], j), rbuf, h + 1, j,
                           send_r, recv_r, right)
                    d.start()
                    started.append(d)
            if h < N_L:
                for j in range(SUBS):
                    mk(sub(lbuf.at[h], j), lbuf, h, j,
                       send_l, recv_l, left).wait_recv()
                    if h + 1 < N_L:
                        d = mk(sub(lbuf.at[h], j), lbuf, h + 1, j,
                               send_l, recv_l, left)
                        d.start()
                        started.append(d)
            compute(rbuf.at[h], p_ref[lax.rem(q + (N_DEV - 1 - h), N_DEV)])
            if h < N_L:
                compute(lbuf.at[h], p_ref[lax.rem(q + (1 + h), N_DEV)])

        for d in last_store:
            if d is not None:
                d.wait()
        for d in started:
            d.wait_send()

    out_shape = jax.ShapeDtypeStruct((N_DEV * m_per, n_per), jnp.float32)
    return pl.pallas_call(
        body,
        out_shape=out_shape,
        in_specs=[
            pl.BlockSpec(memory_space=pl.ANY),
            pl.BlockSpec(memory_space=pl.ANY),
            pl.BlockSpec(memory_space=pltpu.SMEM),
            pl.BlockSpec(memory_space=pltpu.SMEM),
            pl.BlockSpec(memory_space=pltpu.SMEM),
            pl.BlockSpec(memory_space=pltpu.SMEM),
        ],
        out_specs=pl.BlockSpec(memory_space=pl.ANY),
        scratch_shapes=[
            pltpu.VMEM((m_per, k), jnp.float32),
            pltpu.VMEM((k, n_per), jnp.float32),
            pltpu.VMEM((m_per, k), jnp.float8_e5m2),
            pltpu.VMEM((k, n_per), jnp.float8_e5m2),
            pltpu.VMEM((N_R, m_per, k), jnp.float8_e5m2),
            pltpu.VMEM((N_L, m_per, k), jnp.float8_e5m2),
            pltpu.VMEM((N_SLAB, m_per, n_per), jnp.float32),
            pltpu.SemaphoreType.DMA((2,)),
            pltpu.SemaphoreType.DMA((N_SLAB,)),
            pltpu.SemaphoreType.DMA((N_R, SUBS)),
            pltpu.SemaphoreType.DMA((N_L, SUBS)),
            pltpu.SemaphoreType.DMA((N_R, SUBS)),
            pltpu.SemaphoreType.DMA((N_L, SUBS)),
        ],
        compiler_params=pltpu.CompilerParams(
            collective_id=0, vmem_limit_bytes=64 * 1024 * 1024),
    )(x, w_mat, scale_x.reshape(1, 1), scale_w.reshape(1, 1),
      jnp.asarray(P_RING, jnp.int32), jnp.asarray(Q_RING, jnp.int32))
